# baseline (device time: 24915 ns/iter reference)
import jax
import jax.numpy as jnp
from jax import lax
from jax.experimental import pallas as pl
from jax.experimental.pallas import tpu as pltpu

N_DEV = 4
B, Sq, Hq, Dh = 2, 128, 4, 64
BLK = 64


def kernel(x, Wq, K_ext, V_ext, Wo):
    skv_loc = K_ext.shape[1]
    d_model = x.shape[-1]
    dq = Hq * Dh

    K2 = K_ext.reshape(B, skv_loc, dq)
    V2 = V_ext.reshape(B, skv_loc, dq)

    def body(x_ref, wq_ref, k_ref, v_ref, wo_ref, out_ref,
             kv_buf, send_sems, recv_sems, copy_sems, exit_sem):
        my = lax.axis_index("i")

        barrier = pltpu.get_barrier_semaphore()

        @pl.when(my == 0)
        def _():
            for d in range(1, N_DEV):
                pl.semaphore_signal(
                    barrier, inc=1, device_id=(d,),
                    device_id_type=pl.DeviceIdType.MESH)
            pl.semaphore_wait(barrier, N_DEV - 1)

        @pl.when(my != 0)
        def _():
            pl.semaphore_signal(
                barrier, inc=1, device_id=(0,),
                device_id_type=pl.DeviceIdType.MESH)
            pl.semaphore_wait(barrier, 1)

        sends = []
        for idx, d in enumerate(range(1, N_DEV)):
            for s, src in enumerate((k_ref, v_ref)):
                sends.append(pltpu.make_async_remote_copy(
                    src_ref=src,
                    dst_ref=kv_buf.at[s],
                    send_sem=send_sems.at[2 * idx + s],
                    recv_sem=recv_sems.at[s],
                    device_id=(d,),
                    device_id_type=pl.DeviceIdType.MESH,
                ))
        copy_k = pltpu.make_async_copy(k_ref, kv_buf.at[0], copy_sems.at[0])
        copy_v = pltpu.make_async_copy(v_ref, kv_buf.at[1], copy_sems.at[1])
        recv_k = pltpu.make_async_remote_copy(
            src_ref=kv_buf.at[0], dst_ref=kv_buf.at[0],
            send_sem=send_sems.at[0], recv_sem=recv_sems.at[0],
            device_id=(0,), device_id_type=pl.DeviceIdType.MESH)
        recv_v = pltpu.make_async_remote_copy(
            src_ref=kv_buf.at[1], dst_ref=kv_buf.at[1],
            send_sem=send_sems.at[1], recv_sem=recv_sems.at[1],
            device_id=(0,), device_id_type=pl.DeviceIdType.MESH)

        @pl.when(my == 0)
        def _():
            for r in sends:
                r.start()
            copy_k.start()
            copy_v.start()

        wq = wq_ref[...]
        q_alls = [
            jnp.dot(x_ref[b], wq, preferred_element_type=jnp.float32) * 0.125
            for b in range(B)
        ]

        @pl.when(my == 0)
        def _():
            copy_k.wait()
            copy_v.wait()

        @pl.when(my != 0)
        def _():
            recv_k.wait_recv()
            recv_v.wait_recv()

        wo = wo_ref[...]
        for b in range(B):
            q_all = q_alls[b]
            ctx_heads = []
            for h in range(Hq):
                k_bh = kv_buf[0, b, :, h * Dh:(h + 1) * Dh]
                v_bh = kv_buf[1, b, :, h * Dh:(h + 1) * Dh]
                q0 = q_all[0:BLK, h * Dh:(h + 1) * Dh]
                q1 = q_all[BLK:Sq, h * Dh:(h + 1) * Dh]
                s0 = lax.dot_general(
                    q0, k_bh[0:BLK, :], (((1,), (1,)), ((), ())),
                    preferred_element_type=jnp.float32)
                s1 = lax.dot_general(
                    q1, k_bh, (((1,), (1,)), ((), ())),
                    preferred_element_type=jnp.float32)
                w0 = jnp.exp(s0 - jnp.max(s0, axis=1, keepdims=True))
                w0 = w0 / jnp.sum(w0, axis=1, keepdims=True)
                w1 = jnp.exp(s1 - jnp.max(s1, axis=1, keepdims=True))
                w1 = w1 / jnp.sum(w1, axis=1, keepdims=True)
                c0 = jnp.dot(w0, v_bh[0:BLK, :],
                             preferred_element_type=jnp.float32)
                c1 = jnp.dot(w1, v_bh,
                             preferred_element_type=jnp.float32)
                ctx_heads.append(jnp.concatenate([c0, c1], axis=0))
            ctx = jnp.concatenate(ctx_heads, axis=1)
            out_ref[b] = jnp.dot(ctx, wo,
                                 preferred_element_type=jnp.float32)

        @pl.when(my == 0)
        def _():
            for r in sends:
                r.wait_send()
            pl.semaphore_wait(exit_sem, N_DEV - 1)

        @pl.when(my != 0)
        def _():
            pl.semaphore_signal(
                exit_sem, inc=1, device_id=(0,),
                device_id_type=pl.DeviceIdType.MESH)

    return pl.pallas_call(
        body,
        out_shape=jax.ShapeDtypeStruct((B, Sq, d_model), jnp.float32),
        in_specs=[pl.BlockSpec(memory_space=pltpu.VMEM)] * 5,
        out_specs=pl.BlockSpec(memory_space=pltpu.VMEM),
        scratch_shapes=[
            pltpu.VMEM((2, B, skv_loc, dq), jnp.float32),
            pltpu.SemaphoreType.DMA((2 * (N_DEV - 1),)),
            pltpu.SemaphoreType.DMA((2,)),
            pltpu.SemaphoreType.DMA((2,)),
            pltpu.SemaphoreType.REGULAR,
        ],
        compiler_params=pltpu.CompilerParams(collective_id=0),
    )(x, Wq, K2, V2, Wo)


# device time: 20790 ns/iter; 1.1984x vs baseline; 1.1984x over previous
import jax
import jax.numpy as jnp
from jax import lax
from jax.experimental import pallas as pl
from jax.experimental.pallas import tpu as pltpu

N_DEV = 4
B, Sq, Hq, Dh = 2, 128, 4, 64
BLK = 64


def kernel(x, Wq, K_ext, V_ext, Wo):
    skv_loc = K_ext.shape[1]
    d_model = x.shape[-1]
    dq = Hq * Dh

    K2 = K_ext.reshape(B, skv_loc, dq)
    V2 = V_ext.reshape(B, skv_loc, dq)

    def body(x_ref, wq_ref, k_ref, v_ref, wo_ref, out_ref,
             kv_buf, send_sems, fwd_sems, recv_sems, copy_sems, exit_sem):
        my = lax.axis_index("i")

        barrier = pltpu.get_barrier_semaphore()

        def _sig(d):
            pl.semaphore_signal(barrier, inc=1, device_id=(d,),
                                device_id_type=pl.DeviceIdType.MESH)

        @pl.when(my == 0)
        def _():
            _sig(1); _sig(3)
            pl.semaphore_wait(barrier, 2)

        @pl.when(my == 1)
        def _():
            _sig(0); _sig(2)
            pl.semaphore_wait(barrier, 2)

        @pl.when(my == 2)
        def _():
            _sig(1)
            pl.semaphore_wait(barrier, 1)

        @pl.when(my == 3)
        def _():
            _sig(0)
            pl.semaphore_wait(barrier, 1)

        def chunk_src(c):
            b, s = divmod(c, 2)
            return (k_ref if s == 0 else v_ref).at[b]

        def chunk_dst(c):
            b, s = divmod(c, 2)
            return kv_buf.at[s, b]

        sends = []
        for c in range(4):
            for j, d in enumerate((1, 3)):
                sends.append(pltpu.make_async_remote_copy(
                    src_ref=chunk_src(c),
                    dst_ref=chunk_dst(c),
                    send_sem=send_sems.at[2 * c + j],
                    recv_sem=recv_sems.at[c],
                    device_id=(d,),
                    device_id_type=pl.DeviceIdType.MESH,
                ))
        fwds = [pltpu.make_async_remote_copy(
            src_ref=chunk_dst(c), dst_ref=chunk_dst(c),
            send_sem=fwd_sems.at[c], recv_sem=recv_sems.at[c],
            device_id=(2,), device_id_type=pl.DeviceIdType.MESH,
        ) for c in range(4)]
        recvs = [pltpu.make_async_remote_copy(
            src_ref=chunk_dst(c), dst_ref=chunk_dst(c),
            send_sem=fwd_sems.at[c], recv_sem=recv_sems.at[c],
            device_id=(0,), device_id_type=pl.DeviceIdType.MESH,
        ) for c in range(4)]
        copy_k = pltpu.make_async_copy(k_ref, kv_buf.at[0], copy_sems.at[0])
        copy_v = pltpu.make_async_copy(v_ref, kv_buf.at[1], copy_sems.at[1])

        @pl.when(my == 0)
        def _():
            for r in sends:
                r.start()
            copy_k.start()
            copy_v.start()

        wq = wq_ref[...]
        q_alls = [
            jnp.dot(x_ref[b], wq, preferred_element_type=jnp.float32) * 0.125
            for b in range(B)
        ]

        @pl.when(my == 0)
        def _():
            copy_k.wait()
            copy_v.wait()

        wo = wo_ref[...]
        for b in range(B):
            for s in (0, 1):
                c = 2 * b + s

                @pl.when(my != 0)
                def _(c=c):
                    recvs[c].wait_recv()

                @pl.when(my == 1)
                def _(c=c):
                    fwds[c].start()

            q_all = q_alls[b]
            ctx_heads = []
            for h in range(Hq):
                k_bh = kv_buf[0, b, :, h * Dh:(h + 1) * Dh]
                v_bh = kv_buf[1, b, :, h * Dh:(h + 1) * Dh]
                q0 = q_all[0:BLK, h * Dh:(h + 1) * Dh]
                q1 = q_all[BLK:Sq, h * Dh:(h + 1) * Dh]
                s0 = lax.dot_general(
                    q0, k_bh[0:BLK, :], (((1,), (1,)), ((), ())),
                    preferred_element_type=jnp.float32)
                s1 = lax.dot_general(
                    q1, k_bh, (((1,), (1,)), ((), ())),
                    preferred_element_type=jnp.float32)
                w0 = jnp.exp(s0 - jnp.max(s0, axis=1, keepdims=True))
                w0 = w0 / jnp.sum(w0, axis=1, keepdims=True)
                w1 = jnp.exp(s1 - jnp.max(s1, axis=1, keepdims=True))
                w1 = w1 / jnp.sum(w1, axis=1, keepdims=True)
                c0 = jnp.dot(w0, v_bh[0:BLK, :],
                             preferred_element_type=jnp.float32)
                c1 = jnp.dot(w1, v_bh,
                             preferred_element_type=jnp.float32)
                ctx_heads.append(jnp.concatenate([c0, c1], axis=0))
            ctx = jnp.concatenate(ctx_heads, axis=1)
            out_ref[b] = jnp.dot(ctx, wo,
                                 preferred_element_type=jnp.float32)

        @pl.when(my == 0)
        def _():
            for r in sends:
                r.wait_send()
            pl.semaphore_wait(exit_sem, N_DEV - 1)

        @pl.when(my == 1)
        def _():
            for f in fwds:
                f.wait_send()

        @pl.when(my != 0)
        def _():
            pl.semaphore_signal(
                exit_sem, inc=1, device_id=(0,),
                device_id_type=pl.DeviceIdType.MESH)

    return pl.pallas_call(
        body,
        out_shape=jax.ShapeDtypeStruct((B, Sq, d_model), jnp.float32),
        in_specs=[pl.BlockSpec(memory_space=pltpu.VMEM)] * 5,
        out_specs=pl.BlockSpec(memory_space=pltpu.VMEM),
        scratch_shapes=[
            pltpu.VMEM((2, B, skv_loc, dq), jnp.float32),
            pltpu.SemaphoreType.DMA((8,)),
            pltpu.SemaphoreType.DMA((4,)),
            pltpu.SemaphoreType.DMA((4,)),
            pltpu.SemaphoreType.DMA((2,)),
            pltpu.SemaphoreType.REGULAR,
        ],
        compiler_params=pltpu.CompilerParams(collective_id=0),
    )(x, Wq, K2, V2, Wo)


# device time: 17264 ns/iter; 1.4432x vs baseline; 1.2042x over previous
import jax
import jax.numpy as jnp
from jax import lax
from jax.experimental import pallas as pl
from jax.experimental.pallas import tpu as pltpu

N_DEV = 4
B, Sq, Hq, Dh = 2, 128, 4, 64
BLK = 64


def kernel(x, Wq, K_ext, V_ext, Wo):
    skv_loc = K_ext.shape[1]
    d_model = x.shape[-1]
    dq = Hq * Dh

    K2 = K_ext.reshape(B, skv_loc, dq)
    V2 = V_ext.reshape(B, skv_loc, dq)

    def body(x_ref, wq_ref, k_ref, v_ref, wo_ref, out_ref,
             kv_buf, send_sems, fwd_sems, recv_sems, copy_sems):
        my = lax.axis_index("i")

        barrier = pltpu.get_barrier_semaphore()

        def _sig(d):
            pl.semaphore_signal(barrier, inc=1, device_id=(d,),
                                device_id_type=pl.DeviceIdType.MESH)

        @pl.when((my == 1) | (my == 3))
        def _():
            _sig(0)

        @pl.when(my == 2)
        def _():
            _sig(1)

        @pl.when(my == 0)
        def _():
            pl.semaphore_wait(barrier, 2)

        @pl.when(my == 1)
        def _():
            pl.semaphore_wait(barrier, 1)

        def chunk_src(c):
            b, s = divmod(c, 2)
            return (k_ref if s == 0 else v_ref).at[b]

        def chunk_dst(c):
            b, s = divmod(c, 2)
            return kv_buf.at[s, b]

        sends = []
        for c in range(4):
            for j, d in enumerate((1, 3)):
                sends.append(pltpu.make_async_remote_copy(
                    src_ref=chunk_src(c),
                    dst_ref=chunk_dst(c),
                    send_sem=send_sems.at[2 * c + j],
                    recv_sem=recv_sems.at[c],
                    device_id=(d,),
                    device_id_type=pl.DeviceIdType.MESH,
                ))
        fwds = [pltpu.make_async_remote_copy(
            src_ref=chunk_dst(c), dst_ref=chunk_dst(c),
            send_sem=fwd_sems.at[c], recv_sem=recv_sems.at[c],
            device_id=(2,), device_id_type=pl.DeviceIdType.MESH,
        ) for c in range(4)]
        recvs = [pltpu.make_async_remote_copy(
            src_ref=chunk_dst(c), dst_ref=chunk_dst(c),
            send_sem=fwd_sems.at[c], recv_sem=recv_sems.at[c],
            device_id=(0,), device_id_type=pl.DeviceIdType.MESH,
        ) for c in range(4)]
        copy_k = pltpu.make_async_copy(k_ref, kv_buf.at[0], copy_sems.at[0])
        copy_v = pltpu.make_async_copy(v_ref, kv_buf.at[1], copy_sems.at[1])

        @pl.when(my == 0)
        def _():
            for r in sends:
                r.start()
            copy_k.start()
            copy_v.start()

        wq = wq_ref[...]
        q_alls = [
            jnp.dot(x_ref[b], wq, preferred_element_type=jnp.float32) * 0.125
            for b in range(B)
        ]

        @pl.when(my == 0)
        def _():
            copy_k.wait()
            copy_v.wait()

        wo = wo_ref[...]
        for b in range(B):
            for s in (0, 1):
                c = 2 * b + s

                @pl.when(my != 0)
                def _(c=c):
                    recvs[c].wait_recv()

                @pl.when(my == 1)
                def _(c=c):
                    fwds[c].start()

            q_all = q_alls[b]
            ctx_heads = []
            for h in range(Hq):
                k_bh = kv_buf[0, b, :, h * Dh:(h + 1) * Dh]
                v_bh = kv_buf[1, b, :, h * Dh:(h + 1) * Dh]
                q0 = q_all[0:BLK, h * Dh:(h + 1) * Dh]
                q1 = q_all[BLK:Sq, h * Dh:(h + 1) * Dh]
                s0 = lax.dot_general(
                    q0, k_bh[0:BLK, :], (((1,), (1,)), ((), ())),
                    preferred_element_type=jnp.float32)
                s1 = lax.dot_general(
                    q1, k_bh, (((1,), (1,)), ((), ())),
                    preferred_element_type=jnp.float32)
                w0 = jnp.exp(s0 - jnp.max(s0, axis=1, keepdims=True))
                w0 = w0 / jnp.sum(w0, axis=1, keepdims=True)
                w1 = jnp.exp(s1 - jnp.max(s1, axis=1, keepdims=True))
                w1 = w1 / jnp.sum(w1, axis=1, keepdims=True)
                c0 = jnp.dot(w0, v_bh[0:BLK, :],
                             preferred_element_type=jnp.float32)
                c1 = jnp.dot(w1, v_bh,
                             preferred_element_type=jnp.float32)
                ctx_heads.append(jnp.concatenate([c0, c1], axis=0))
            ctx = jnp.concatenate(ctx_heads, axis=1)
            out_ref[b] = jnp.dot(ctx, wo,
                                 preferred_element_type=jnp.float32)

        @pl.when(my == 0)
        def _():
            for r in sends:
                r.wait_send()

        @pl.when(my == 1)
        def _():
            for f in fwds:
                f.wait_send()

    return pl.pallas_call(
        body,
        out_shape=jax.ShapeDtypeStruct((B, Sq, d_model), jnp.float32),
        in_specs=[pl.BlockSpec(memory_space=pltpu.VMEM)] * 5,
        out_specs=pl.BlockSpec(memory_space=pltpu.VMEM),
        scratch_shapes=[
            pltpu.VMEM((2, B, skv_loc, dq), jnp.float32),
            pltpu.SemaphoreType.DMA((8,)),
            pltpu.SemaphoreType.DMA((4,)),
            pltpu.SemaphoreType.DMA((4,)),
            pltpu.SemaphoreType.DMA((2,)),
        ],
        compiler_params=pltpu.CompilerParams(collective_id=0),
    )(x, Wq, K2, V2, Wo)


# device time: 15203 ns/iter; 1.6388x vs baseline; 1.1356x over previous
import jax
import jax.numpy as jnp
from jax import lax
from jax.experimental import pallas as pl
from jax.experimental.pallas import tpu as pltpu

N_DEV = 4
B, Sq, Hq, Dh = 2, 128, 4, 64
BLK = 64


def kernel(x, Wq, K_ext, V_ext, Wo):
    skv_loc = K_ext.shape[1]
    d_model = x.shape[-1]
    dq = Hq * Dh

    def body(x_ref, wq_ref, k_ref, v_ref, wo_ref, out_ref,
             kv_buf, send_sems, fwd_sems, recv_sems):
        my = lax.axis_index("i")

        barrier = pltpu.get_barrier_semaphore()

        def _sig(d):
            pl.semaphore_signal(barrier, inc=1, device_id=(d,),
                                device_id_type=pl.DeviceIdType.MESH)

        @pl.when((my == 1) | (my == 3))
        def _():
            _sig(0)

        @pl.when(my == 2)
        def _():
            _sig(1)
            _sig(3)

        @pl.when(my == 0)
        def _():
            pl.semaphore_wait(barrier, 2)

        @pl.when((my == 1) | (my == 3))
        def _():
            pl.semaphore_wait(barrier, 1)

        def chunk_dst(c):
            b, s = divmod(c, 2)
            return kv_buf.at[s, b]

        def mk_send(c, d, i):
            return pltpu.make_async_remote_copy(
                src_ref=chunk_dst(c), dst_ref=chunk_dst(c),
                send_sem=send_sems.at[i], recv_sem=recv_sems.at[c],
                device_id=(d,), device_id_type=pl.DeviceIdType.MESH)

        sends_b = [
            [mk_send(2 * b + 0, 1, 4 * b + 0),
             mk_send(2 * b + 1, 3, 4 * b + 1),
             mk_send(2 * b + 1, 1, 4 * b + 2),
             mk_send(2 * b + 0, 3, 4 * b + 3)]
            for b in range(B)
        ]
        fwd_k = [pltpu.make_async_remote_copy(
            src_ref=chunk_dst(2 * b), dst_ref=chunk_dst(2 * b),
            send_sem=fwd_sems.at[2 * b], recv_sem=recv_sems.at[2 * b],
            device_id=(2,), device_id_type=pl.DeviceIdType.MESH,
        ) for b in range(B)]
        fwd_v = [pltpu.make_async_remote_copy(
            src_ref=chunk_dst(2 * b + 1), dst_ref=chunk_dst(2 * b + 1),
            send_sem=fwd_sems.at[2 * b + 1], recv_sem=recv_sems.at[2 * b + 1],
            device_id=(2,), device_id_type=pl.DeviceIdType.MESH,
        ) for b in range(B)]
        recvs = [pltpu.make_async_remote_copy(
            src_ref=chunk_dst(c), dst_ref=chunk_dst(c),
            send_sem=fwd_sems.at[c], recv_sem=recv_sems.at[c],
            device_id=(0,), device_id_type=pl.DeviceIdType.MESH,
        ) for c in range(4)]

        @pl.when(my == 0)
        def _():
            for b in range(B):
                kv_buf[0, b] = jnp.reshape(k_ref[b], (skv_loc, dq))
                kv_buf[1, b] = jnp.reshape(v_ref[b], (skv_loc, dq))
                for r in sends_b[b]:
                    r.start()

        wq = wq_ref[...]
        q_alls = [
            jnp.dot(x_ref[b], wq, preferred_element_type=jnp.float32) * 0.125
            for b in range(B)
        ]

        wo = wo_ref[...]
        for b in range(B):
            c_k, c_v = 2 * b, 2 * b + 1

            @pl.when(my == 1)
            def _(b=b, c_k=c_k, c_v=c_v):
                recvs[c_k].wait_recv()
                fwd_k[b].start()
                recvs[c_v].wait_recv()

            @pl.when(my == 3)
            def _(b=b, c_k=c_k, c_v=c_v):
                recvs[c_v].wait_recv()
                fwd_v[b].start()
                recvs[c_k].wait_recv()

            @pl.when(my == 2)
            def _(c_k=c_k, c_v=c_v):
                recvs[c_k].wait_recv()
                recvs[c_v].wait_recv()

            q_all = q_alls[b]
            ctx_heads = []
            for h in range(Hq):
                k_bh = kv_buf[0, b, :, h * Dh:(h + 1) * Dh]
                v_bh = kv_buf[1, b, :, h * Dh:(h + 1) * Dh]
                q0 = q_all[0:BLK, h * Dh:(h + 1) * Dh]
                q1 = q_all[BLK:Sq, h * Dh:(h + 1) * Dh]
                s0 = lax.dot_general(
                    q0, k_bh[0:BLK, :], (((1,), (1,)), ((), ())),
                    preferred_element_type=jnp.float32)
                s1 = lax.dot_general(
                    q1, k_bh, (((1,), (1,)), ((), ())),
                    preferred_element_type=jnp.float32)
                w0 = jnp.exp(s0 - jnp.max(s0, axis=1, keepdims=True))
                w0 = w0 / jnp.sum(w0, axis=1, keepdims=True)
                w1 = jnp.exp(s1 - jnp.max(s1, axis=1, keepdims=True))
                w1 = w1 / jnp.sum(w1, axis=1, keepdims=True)
                c0 = jnp.dot(w0, v_bh[0:BLK, :],
                             preferred_element_type=jnp.float32)
                c1 = jnp.dot(w1, v_bh,
                             preferred_element_type=jnp.float32)
                ctx_heads.append(jnp.concatenate([c0, c1], axis=0))
            ctx = jnp.concatenate(ctx_heads, axis=1)
            out_ref[b] = jnp.dot(ctx, wo,
                                 preferred_element_type=jnp.float32)

        @pl.when(my == 0)
        def _():
            for bs in sends_b:
                for r in bs:
                    r.wait_send()

        @pl.when(my == 1)
        def _():
            for f in fwd_k:
                f.wait_send()

        @pl.when(my == 3)
        def _():
            for f in fwd_v:
                f.wait_send()

    return pl.pallas_call(
        body,
        out_shape=jax.ShapeDtypeStruct((B, Sq, d_model), jnp.float32),
        in_specs=[pl.BlockSpec(memory_space=pltpu.VMEM)] * 5,
        out_specs=pl.BlockSpec(memory_space=pltpu.VMEM),
        scratch_shapes=[
            pltpu.VMEM((2, B, skv_loc, dq), jnp.float32),
            pltpu.SemaphoreType.DMA((8,)),
            pltpu.SemaphoreType.DMA((4,)),
            pltpu.SemaphoreType.DMA((4,)),
        ],
        compiler_params=pltpu.CompilerParams(collective_id=0),
    )(x, Wq, K_ext, V_ext, Wo)
